# baseline (speedup 1.0000x reference)
import jax
import jax.numpy as jnp
from jax import lax
from jax.experimental import pallas as pl
from jax.experimental.pallas import tpu as pltpu
from jax.experimental.pallas import tpu_sc as plsc

N = 10000
N_PAD = 10240
E = 320000
IN_CH = 128
HID = 64
HID2 = 32
OUT = 2
D1 = 72

CHUNK = 128
NCORES = 2
TILES = 16
NW = NCORES * TILES
NCHUNK = E // CHUNK
CPT = 78
NTAIL = NCHUNK - NW * CPT
ROWS_PER_TILE = N_PAD // TILES


def _make_seg_sum(d, K, stage=False):
  G = CPT // K
  out_types = [jax.ShapeDtypeStruct((NCORES, N_PAD, d), jnp.float32)]
  scratch = [
      pltpu.VMEM_SHARED((N_PAD, d), jnp.float32),
      pltpu.VMEM((K, CHUNK), jnp.int32),
      pltpu.VMEM((K, CHUNK), jnp.int32),
      pltpu.VMEM((K, CHUNK), jnp.int32),
      pltpu.VMEM((K, CHUNK), jnp.int32),
      pltpu.VMEM((K, CHUNK, d), jnp.float32),
      pltpu.VMEM((K, CHUNK, d), jnp.float32),
      pltpu.SemaphoreType.DMA,
      pltpu.SemaphoreType.DMA,
      pltpu.SemaphoreType.DMA,
      pltpu.SemaphoreType.DMA,
  ]
  if stage:
    scratch.append(pltpu.VMEM_SHARED((N_PAD, d), jnp.float32))

  mesh = plsc.VectorSubcoreMesh(core_axis_name="c", subcore_axis_name="s")

  def body(vals_hbm, edges_hbm, zeros_hbm, out_hbm,
           acc, srcb0, srcb1, dstb0, dstb1, rows0, rows1,
           sg0, sg1, ss0, ss1, *maybe_spm):
    c = lax.axis_index("c")
    s = lax.axis_index("s")
    w = c * TILES + s
    r0 = s * ROWS_PER_TILE
    srcb = (srcb0, srcb1)
    dstb = (dstb0, dstb1)
    rows = (rows0, rows1)
    sg = (sg0, sg1)
    ss = (ss0, ss1)

    pltpu.sync_copy(zeros_hbm.at[pl.ds(r0, ROWS_PER_TILE)],
                    acc.at[pl.ds(r0, ROWS_PER_TILE)])
    if stage:
      pltpu.sync_copy(vals_hbm.at[pl.ds(r0, ROWS_PER_TILE)],
                      maybe_spm[0].at[pl.ds(r0, ROWS_PER_TILE)])
      vals = maybe_spm[0]
    else:
      vals = vals_hbm
    plsc.subcore_barrier()

    base = w * CPT

    def issue(g, sl):
      row = base + g * K
      pltpu.sync_copy(edges_hbm.at[0, pl.ds(row, K)], srcb[sl])
      pltpu.sync_copy(edges_hbm.at[1, pl.ds(row, K)], dstb[sl])
      for k in range(K):
        pltpu.async_copy(vals.at[srcb[sl].at[k]], rows[sl].at[k], sg[sl])

    def drain_g(sl):
      for k in range(K):
        pltpu.make_async_copy(vals.at[srcb[sl].at[k]], rows[sl].at[k],
                              sg[sl]).wait()

    def fire_s(sl):
      for k in range(K):
        pltpu.async_copy(rows[sl].at[k], acc.at[dstb[sl].at[k]], ss[sl],
                         add=True)

    def drain_s(sl):
      for k in range(K):
        pltpu.make_async_copy(rows[sl].at[k], acc.at[dstb[sl].at[k]],
                              ss[sl]).wait()

    issue(0, 0)
    issue(1, 1)

    def step(i, carry):
      drain_g(0)
      fire_s(0)
      drain_g(1)
      fire_s(1)
      drain_s(0)
      issue(2 * i + 2, 0)
      drain_s(1)
      issue(2 * i + 3, 1)
      return carry

    lax.fori_loop(0, (G - 2) // 2, step, 0)
    drain_g(0)
    fire_s(0)
    drain_g(1)
    fire_s(1)
    if G % 2:
      drain_s(0)
      issue(G - 1, 0)
      drain_g(0)
      fire_s(0)
    drain_s(0)
    drain_s(1)

    @pl.when(w < NTAIL)
    def _tail():
      row = NW * CPT + w
      pltpu.sync_copy(edges_hbm.at[0, pl.ds(row, 1)], srcb0.at[pl.ds(0, 1)])
      pltpu.sync_copy(edges_hbm.at[1, pl.ds(row, 1)], dstb0.at[pl.ds(0, 1)])
      pltpu.sync_copy(vals.at[srcb0.at[0]], rows0.at[0])
      pltpu.sync_copy(rows0.at[0], acc.at[dstb0.at[0]], add=True)

    plsc.subcore_barrier()
    pltpu.sync_copy(acc.at[pl.ds(r0, ROWS_PER_TILE)],
                    out_hbm.at[c, pl.ds(r0, ROWS_PER_TILE)])

  return pl.kernel(
      body, out_type=out_types, mesh=mesh, scratch_types=scratch,
      compiler_params=pltpu.CompilerParams(use_tc_tiling_on_sc=False))


_seg1 = _make_seg_sum(D1, 3)
_seg2 = _make_seg_sum(HID2, 6, stage=True)

_R = 2048


def _tc1_body(x_ref, wl_ref, wr_ref, b_ref, xl_ref, xr_ref):
  xb = x_ref[...]
  dn = (((1,), (1,)), ((), ()))
  xl_ref[:, :HID] = lax.dot_general(xb, wl_ref[...], dn,
                                    preferred_element_type=jnp.float32)
  xl_ref[:, HID:] = jnp.ones((_R, D1 - HID), jnp.float32)
  xr_ref[...] = lax.dot_general(xb, wr_ref[...], dn,
                                preferred_element_type=jnp.float32) + b_ref[...]


def _tc1(x, W1l, W1r, b1):
  return pl.pallas_call(
      _tc1_body,
      grid=(N_PAD // _R,),
      in_specs=[
          pl.BlockSpec((_R, IN_CH), lambda i: (i, 0)),
          pl.BlockSpec((HID, IN_CH), lambda i: (0, 0)),
          pl.BlockSpec((HID, IN_CH), lambda i: (0, 0)),
          pl.BlockSpec((1, HID), lambda i: (0, 0)),
      ],
      out_specs=[
          pl.BlockSpec((_R, D1), lambda i: (i, 0)),
          pl.BlockSpec((_R, HID), lambda i: (i, 0)),
      ],
      out_shape=[
          jax.ShapeDtypeStruct((N_PAD, D1), jnp.float32),
          jax.ShapeDtypeStruct((N_PAD, HID), jnp.float32),
      ],
  )(x, W1l, W1r, b1.reshape(1, HID))


def _tc2_body(p_ref, xr_ref, wl_ref, wr_ref, b_ref, hl_ref, hr_ref):
  ssum = p_ref[0, :, :HID] + p_ref[1, :, :HID]
  cnt = jnp.maximum(p_ref[0, :, HID:HID + 1] + p_ref[1, :, HID:HID + 1], 1.0)
  h1 = jnp.maximum(ssum / cnt + xr_ref[...], 0.0)
  dn = (((1,), (1,)), ((), ()))
  hl_ref[...] = lax.dot_general(h1, wl_ref[...], dn,
                                preferred_element_type=jnp.float32)
  hr_ref[...] = lax.dot_general(h1, wr_ref[...], dn,
                                preferred_element_type=jnp.float32) + b_ref[...]


def _tc2(p, xr, W2l, b2, W2r):
  return pl.pallas_call(
      _tc2_body,
      grid=(N_PAD // _R,),
      in_specs=[
          pl.BlockSpec((NCORES, _R, D1), lambda i: (0, i, 0)),
          pl.BlockSpec((_R, HID), lambda i: (i, 0)),
          pl.BlockSpec((HID2, HID), lambda i: (0, 0)),
          pl.BlockSpec((HID2, HID), lambda i: (0, 0)),
          pl.BlockSpec((1, HID2), lambda i: (0, 0)),
      ],
      out_specs=[
          pl.BlockSpec((_R, HID2), lambda i: (i, 0)),
          pl.BlockSpec((_R, HID2), lambda i: (i, 0)),
      ],
      out_shape=[
          jax.ShapeDtypeStruct((N_PAD, HID2), jnp.float32),
          jax.ShapeDtypeStruct((N_PAD, HID2), jnp.float32),
      ],
  )(p, xr, W2l, W2r, b2.reshape(1, HID2))


def _tc3_body(q_ref, pc_ref, hr_ref, wo_ref, bo_ref, out_ref):
  ssum = q_ref[0] + q_ref[1]
  cnt = jnp.maximum(pc_ref[0, :, HID:HID + 1] + pc_ref[1, :, HID:HID + 1], 1.0)
  h2 = jnp.maximum(ssum / cnt + hr_ref[...], 0.0)
  dn = (((1,), (1,)), ((), ()))
  logits = lax.dot_general(h2, wo_ref[...], dn,
                           preferred_element_type=jnp.float32) + bo_ref[...]
  m = jnp.max(logits, axis=1, keepdims=True)
  shifted = logits - m
  lse = jnp.log(jnp.sum(jnp.exp(shifted), axis=1, keepdims=True))
  out_ref[...] = shifted - lse


def _tc3(q, p, h1r, Wo, bo):
  return pl.pallas_call(
      _tc3_body,
      grid=(N_PAD // _R,),
      in_specs=[
          pl.BlockSpec((NCORES, _R, HID2), lambda i: (0, i, 0)),
          pl.BlockSpec((NCORES, _R, D1), lambda i: (0, i, 0)),
          pl.BlockSpec((_R, HID2), lambda i: (i, 0)),
          pl.BlockSpec((OUT, HID2), lambda i: (0, 0)),
          pl.BlockSpec((1, OUT), lambda i: (0, 0)),
      ],
      out_specs=pl.BlockSpec((_R, OUT), lambda i: (i, 0)),
      out_shape=jax.ShapeDtypeStruct((N, OUT), jnp.float32),
  )(q, p, h1r, Wo, bo.reshape(1, OUT))


def kernel(x, edge_index, W1l, b1, W1r, W2l, b2, W2r, Wo, bo):
  edges3d = edge_index.astype(jnp.int32).reshape(2, NCHUNK, CHUNK)
  zeros1 = jnp.zeros((N_PAD, D1), jnp.float32)
  zeros2 = jnp.zeros((N_PAD, HID2), jnp.float32)

  xl, xr = _tc1(x, W1l, W1r, b1)
  (p,) = _seg1(xl, edges3d, zeros1)
  h1l, h1r = _tc2(p, xr, W2l, b2, W2r)
  (q,) = _seg2(h1l, edges3d, zeros2)
  return _tc3(q, p, h1r, Wo, bo)


# speedup vs baseline: 1.0073x; 1.0073x over previous
import jax
import jax.numpy as jnp
from jax import lax
from jax.experimental import pallas as pl
from jax.experimental.pallas import tpu as pltpu
from jax.experimental.pallas import tpu_sc as plsc

N = 10000
N_PAD = 10240
E = 320000
IN_CH = 128
HID = 64
HID2 = 32
OUT = 2
D1 = 72

CHUNK = 128
NCORES = 2
TILES = 16
NW = NCORES * TILES
NCHUNK = E // CHUNK
CPT = 78
NTAIL = NCHUNK - NW * CPT
ROWS_PER_TILE = N_PAD // TILES


def _make_seg_sum(d, K):
  G = CPT // K
  out_types = [jax.ShapeDtypeStruct((NCORES, N_PAD, d), jnp.float32)]
  scratch = [
      pltpu.VMEM_SHARED((N_PAD, d), jnp.float32),
      pltpu.VMEM((K, CHUNK), jnp.int32),
      pltpu.VMEM((K, CHUNK), jnp.int32),
      pltpu.VMEM((K, CHUNK), jnp.int32),
      pltpu.VMEM((K, CHUNK), jnp.int32),
      pltpu.VMEM((K, CHUNK, d), jnp.float32),
      pltpu.VMEM((K, CHUNK, d), jnp.float32),
      pltpu.SemaphoreType.DMA,
      pltpu.SemaphoreType.DMA,
      pltpu.SemaphoreType.DMA,
      pltpu.SemaphoreType.DMA,
  ]

  mesh = plsc.VectorSubcoreMesh(core_axis_name="c", subcore_axis_name="s")

  def body(vals_hbm, edges_hbm, zeros_hbm, out_hbm,
           acc, srcb0, srcb1, dstb0, dstb1, rows0, rows1,
           sg0, sg1, ss0, ss1):
    c = lax.axis_index("c")
    s = lax.axis_index("s")
    w = c * TILES + s
    r0 = s * ROWS_PER_TILE
    srcb = (srcb0, srcb1)
    dstb = (dstb0, dstb1)
    rows = (rows0, rows1)
    sg = (sg0, sg1)
    ss = (ss0, ss1)

    pltpu.sync_copy(zeros_hbm.at[pl.ds(r0, ROWS_PER_TILE)],
                    acc.at[pl.ds(r0, ROWS_PER_TILE)])
    plsc.subcore_barrier()

    base = w * CPT

    def issue(g, sl):
      row = base + g * K
      pltpu.sync_copy(edges_hbm.at[0, pl.ds(row, K)], srcb[sl])
      pltpu.sync_copy(edges_hbm.at[1, pl.ds(row, K)], dstb[sl])
      for k in range(K):
        pltpu.async_copy(vals_hbm.at[srcb[sl].at[k]], rows[sl].at[k], sg[sl])

    def drain_g(sl):
      for k in range(K):
        pltpu.make_async_copy(vals_hbm.at[srcb[sl].at[k]], rows[sl].at[k],
                              sg[sl]).wait()

    def fire_s(sl):
      for k in range(K):
        pltpu.async_copy(rows[sl].at[k], acc.at[dstb[sl].at[k]], ss[sl],
                         add=True)

    def drain_s(sl):
      for k in range(K):
        pltpu.make_async_copy(rows[sl].at[k], acc.at[dstb[sl].at[k]],
                              ss[sl]).wait()

    issue(0, 0)
    issue(1, 1)

    def step(i, carry):
      drain_g(0)
      fire_s(0)
      drain_g(1)
      fire_s(1)
      drain_s(0)
      issue(2 * i + 2, 0)
      drain_s(1)
      issue(2 * i + 3, 1)
      return carry

    lax.fori_loop(0, (G - 2) // 2, step, 0)
    drain_g(0)
    fire_s(0)
    drain_g(1)
    fire_s(1)
    if G % 2:
      drain_s(0)
      issue(G - 1, 0)
      drain_g(0)
      fire_s(0)
    drain_s(0)
    drain_s(1)

    @pl.when(w < NTAIL)
    def _tail():
      row = NW * CPT + w
      pltpu.sync_copy(edges_hbm.at[0, pl.ds(row, 1)], srcb0.at[pl.ds(0, 1)])
      pltpu.sync_copy(edges_hbm.at[1, pl.ds(row, 1)], dstb0.at[pl.ds(0, 1)])
      pltpu.sync_copy(vals_hbm.at[srcb0.at[0]], rows0.at[0])
      pltpu.sync_copy(rows0.at[0], acc.at[dstb0.at[0]], add=True)

    plsc.subcore_barrier()
    pltpu.sync_copy(acc.at[pl.ds(r0, ROWS_PER_TILE)],
                    out_hbm.at[c, pl.ds(r0, ROWS_PER_TILE)])

  return pl.kernel(
      body, out_type=out_types, mesh=mesh, scratch_types=scratch,
      compiler_params=pltpu.CompilerParams(use_tc_tiling_on_sc=False))


_seg1 = _make_seg_sum(D1, 3)
_seg2 = _make_seg_sum(HID2, 6)

_R = 2048


def _tc1_body(x_ref, wl_ref, wr_ref, b_ref, xl_ref, xr_ref):
  xb = x_ref[...]
  dn = (((1,), (1,)), ((), ()))
  xl_ref[:, :HID] = lax.dot_general(xb, wl_ref[...], dn,
                                    preferred_element_type=jnp.float32)
  xl_ref[:, HID:] = jnp.ones((_R, D1 - HID), jnp.float32)
  xr_ref[...] = lax.dot_general(xb, wr_ref[...], dn,
                                preferred_element_type=jnp.float32) + b_ref[...]


def _tc1(x, W1l, W1r, b1):
  return pl.pallas_call(
      _tc1_body,
      grid=(N_PAD // _R,),
      in_specs=[
          pl.BlockSpec((_R, IN_CH), lambda i: (i, 0)),
          pl.BlockSpec((HID, IN_CH), lambda i: (0, 0)),
          pl.BlockSpec((HID, IN_CH), lambda i: (0, 0)),
          pl.BlockSpec((1, HID), lambda i: (0, 0)),
      ],
      out_specs=[
          pl.BlockSpec((_R, D1), lambda i: (i, 0)),
          pl.BlockSpec((_R, HID), lambda i: (i, 0)),
      ],
      out_shape=[
          jax.ShapeDtypeStruct((N_PAD, D1), jnp.float32),
          jax.ShapeDtypeStruct((N_PAD, HID), jnp.float32),
      ],
  )(x, W1l, W1r, b1.reshape(1, HID))


def _tc2_body(p_ref, xr_ref, wl_ref, wr_ref, b_ref, hl_ref, hr_ref):
  ssum = p_ref[0, :, :HID] + p_ref[1, :, :HID]
  cnt = jnp.maximum(p_ref[0, :, HID:HID + 1] + p_ref[1, :, HID:HID + 1], 1.0)
  h1 = jnp.maximum(ssum / cnt + xr_ref[...], 0.0)
  dn = (((1,), (1,)), ((), ()))
  hl_ref[...] = lax.dot_general(h1, wl_ref[...], dn,
                                preferred_element_type=jnp.float32)
  hr_ref[...] = lax.dot_general(h1, wr_ref[...], dn,
                                preferred_element_type=jnp.float32) + b_ref[...]


def _tc2(p, xr, W2l, b2, W2r):
  return pl.pallas_call(
      _tc2_body,
      grid=(N_PAD // _R,),
      in_specs=[
          pl.BlockSpec((NCORES, _R, D1), lambda i: (0, i, 0)),
          pl.BlockSpec((_R, HID), lambda i: (i, 0)),
          pl.BlockSpec((HID2, HID), lambda i: (0, 0)),
          pl.BlockSpec((HID2, HID), lambda i: (0, 0)),
          pl.BlockSpec((1, HID2), lambda i: (0, 0)),
      ],
      out_specs=[
          pl.BlockSpec((_R, HID2), lambda i: (i, 0)),
          pl.BlockSpec((_R, HID2), lambda i: (i, 0)),
      ],
      out_shape=[
          jax.ShapeDtypeStruct((N_PAD, HID2), jnp.float32),
          jax.ShapeDtypeStruct((N_PAD, HID2), jnp.float32),
      ],
  )(p, xr, W2l, W2r, b2.reshape(1, HID2))


def _tc3_body(q_ref, pc_ref, hr_ref, wo_ref, bo_ref, out_ref):
  ssum = q_ref[0] + q_ref[1]
  cnt = jnp.maximum(pc_ref[0, :, HID:HID + 1] + pc_ref[1, :, HID:HID + 1], 1.0)
  h2 = jnp.maximum(ssum / cnt + hr_ref[...], 0.0)
  dn = (((1,), (1,)), ((), ()))
  logits = lax.dot_general(h2, wo_ref[...], dn,
                           preferred_element_type=jnp.float32) + bo_ref[...]
  m = jnp.max(logits, axis=1, keepdims=True)
  shifted = logits - m
  lse = jnp.log(jnp.sum(jnp.exp(shifted), axis=1, keepdims=True))
  out_ref[...] = shifted - lse


def _tc3(q, p, h1r, Wo, bo):
  return pl.pallas_call(
      _tc3_body,
      grid=(N_PAD // _R,),
      in_specs=[
          pl.BlockSpec((NCORES, _R, HID2), lambda i: (0, i, 0)),
          pl.BlockSpec((NCORES, _R, D1), lambda i: (0, i, 0)),
          pl.BlockSpec((_R, HID2), lambda i: (i, 0)),
          pl.BlockSpec((OUT, HID2), lambda i: (0, 0)),
          pl.BlockSpec((1, OUT), lambda i: (0, 0)),
      ],
      out_specs=pl.BlockSpec((_R, OUT), lambda i: (i, 0)),
      out_shape=jax.ShapeDtypeStruct((N, OUT), jnp.float32),
  )(q, p, h1r, Wo, bo.reshape(1, OUT))


def kernel(x, edge_index, W1l, b1, W1r, W2l, b2, W2r, Wo, bo):
  edges3d = edge_index.astype(jnp.int32).reshape(2, NCHUNK, CHUNK)
  zeros1 = jnp.zeros((N_PAD, D1), jnp.float32)
  zeros2 = jnp.zeros((N_PAD, HID2), jnp.float32)

  xl, xr = _tc1(x, W1l, W1r, b1)
  (p,) = _seg1(xl, edges3d, zeros1)
  h1l, h1r = _tc2(p, xr, W2l, b2, W2r)
  (q,) = _seg2(h1l, edges3d, zeros2)
  return _tc3(q, p, h1r, Wo, bo)
